# baseline (speedup 1.0000x reference)
import jax
import jax.numpy as jnp
from jax import lax
from jax.experimental import pallas as pl
from jax.experimental.pallas import tpu as pltpu
from jax.experimental.pallas import tpu_sc as plsc

N = 10000
E = 320000
D_IN = 128
D_OUT = 40
HW = 24
DP = 2 * HW
NC, NS, L = 2, 16, 16
CHUNK = 128
NB = 2
NCHUNKS = E // CHUNK
CPT = 157
CPT_LAST = NCHUNKS - 15 * CPT
NP = 10240
RPT = NP // NS
RLAST = N - 15 * RPT


def _mlp_body(x_ref, w0_ref, w1_ref, o_ref):
    h = jnp.maximum(
        jnp.dot(x_ref[...], w0_ref[...], preferred_element_type=jnp.float32), 0.0)
    h = jnp.dot(h, w1_ref[...], preferred_element_type=jnp.float32)
    o_ref[pl.ds(0, N), :] = jnp.pad(h, ((0, 0), (0, DP - D_OUT)))


_mlp = pl.pallas_call(
    _mlp_body,
    out_shape=jax.ShapeDtypeStruct((NP, DP), jnp.float32),
)


_GDN = lax.GatherDimensionNumbers(
    offset_dims=(), collapsed_slice_dims=(0,), start_index_map=(0,))


def _spmm_body(h_hbm, ei_hbm, vals_hbm, zeros_hbm, out_hbm,
               src_v, dst_v, vals_v, rows_v, stage_v, acc, h_s, gsem, ssem):
    cid = lax.axis_index("c")
    sid = lax.axis_index("s")
    sl = pl.ds(sid * RPT, RPT)
    last = sid == NS - 1
    cpt = jnp.where(last, CPT_LAST, CPT)
    cbase = sid * CPT

    iota = lax.iota(jnp.int32, L)
    duo = iota // 8
    pcol = L + iota - 8 * duo

    pltpu.sync_copy(zeros_hbm, stage_v)
    pltpu.sync_copy(stage_v, acc.at[sl])
    pltpu.sync_copy(h_hbm.at[sl, pl.ds(cid * HW, HW)], stage_v)
    pltpu.sync_copy(stage_v, h_s.at[sl])

    @pl.when(jnp.logical_not(last))
    def _():
        csl = pl.ds(cbase, CPT)
        pltpu.sync_copy(ei_hbm.at[0].at[csl], src_v)
        pltpu.sync_copy(ei_hbm.at[1].at[csl], dst_v)
        pltpu.sync_copy(vals_hbm.at[csl], vals_v)

    @pl.when(last)
    def _():
        csl = pl.ds(cbase, CPT_LAST)
        tsl = pl.ds(0, CPT_LAST)
        pltpu.sync_copy(ei_hbm.at[0].at[csl], src_v.at[tsl])
        pltpu.sync_copy(ei_hbm.at[1].at[csl], dst_v.at[tsl])
        pltpu.sync_copy(vals_hbm.at[csl], vals_v.at[tsl])

    def spmm_round():
        plsc.subcore_barrier()

        for b in range(NB):
            pltpu.async_copy(h_s.at[src_v.at[b]], rows_v.at[b], gsem[b])

        def outer(g, carry):
            for b in range(NB):
                j = g * NB + b

                @pl.when(j < cpt)
                def _():
                    pltpu.make_async_copy(
                        h_s.at[src_v.at[j]], rows_v.at[b], gsem[b]).wait()

                    @plsc.parallel_loop(0, CHUNK, L, unroll=2)
                    def _scale(c0):
                        v16 = vals_v[j, pl.ds(c0, L)]
                        for r in range(L):
                            bc = lax.gather(
                                v16, jnp.full((L, 1), r, jnp.int32), _GDN,
                                slice_sizes=(1,),
                                mode=lax.GatherScatterMode.PROMISE_IN_BOUNDS)
                            seg = rows_v[b, c0 + r, pl.ds(0, L)]
                            rows_v[b, c0 + r, pl.ds(0, L)] = seg * bc
                        for t in range(8):
                            r_idx = (jnp.full((L,), c0 + 2 * t, jnp.int32)
                                     + duo)
                            b_idx = jnp.full((L,), b, jnp.int32)
                            vv = plsc.load_gather(
                                vals_v, [jnp.full((L,), j, jnp.int32), r_idx])
                            seg = plsc.load_gather(
                                rows_v, [b_idx, r_idx, pcol])
                            plsc.store_scatter(rows_v, [b_idx, r_idx, pcol],
                                               seg * vv)

                    pltpu.async_copy(rows_v.at[b], acc.at[dst_v.at[j]],
                                     ssem[b], add=True)

                    b2 = (b - 1) % NB
                    j2 = j - 1
                    jn = j2 + NB

                    @pl.when(j2 >= 0)
                    def _():
                        pltpu.make_async_copy(
                            rows_v.at[b2], acc.at[dst_v.at[j2]],
                            ssem[b2]).wait()

                        @pl.when(jn < cpt)
                        def _():
                            pltpu.async_copy(
                                h_s.at[src_v.at[jn]], rows_v.at[b2],
                                gsem[b2])
            return carry

        lax.fori_loop(0, (CPT + NB - 1) // NB, outer, 0)

        pltpu.make_async_copy(
            rows_v.at[0], acc.at[dst_v.at[cpt - 1]], ssem[0]).wait()

        plsc.subcore_barrier()

    spmm_round()

    pltpu.sync_copy(acc.at[sl], stage_v)
    pltpu.sync_copy(stage_v, h_s.at[sl])
    pltpu.sync_copy(zeros_hbm, stage_v)
    pltpu.sync_copy(stage_v, acc.at[sl])

    spmm_round()

    pltpu.sync_copy(acc.at[sl], stage_v)
    nd = D_OUT - HW

    @pl.when(jnp.logical_not(last))
    def _():
        rs = pl.ds(sid * RPT, RPT)

        @pl.when(cid == 0)
        def _():
            pltpu.sync_copy(stage_v, out_hbm.at[rs, pl.ds(0, HW)])

        @pl.when(cid == 1)
        def _():
            pltpu.sync_copy(stage_v.at[:, pl.ds(0, nd)],
                            out_hbm.at[rs, pl.ds(HW, nd)])

    @pl.when(last)
    def _():
        rs = pl.ds(sid * RPT, RLAST)

        @pl.when(cid == 0)
        def _():
            pltpu.sync_copy(stage_v.at[pl.ds(0, RLAST)],
                            out_hbm.at[rs, pl.ds(0, HW)])

        @pl.when(cid == 1)
        def _():
            pltpu.sync_copy(stage_v.at[pl.ds(0, RLAST), pl.ds(0, nd)],
                            out_hbm.at[rs, pl.ds(HW, nd)])


_spmm = pl.kernel(
    _spmm_body,
    out_type=jax.ShapeDtypeStruct((N, D_OUT), jnp.float32),
    mesh=plsc.VectorSubcoreMesh(core_axis_name="c", subcore_axis_name="s"),
    compiler_params=pltpu.CompilerParams(needs_layout_passes=False,
                                         use_tc_tiling_on_sc=False),
    scratch_types=[
        pltpu.VMEM((CPT, CHUNK), jnp.int32),
        pltpu.VMEM((CPT, CHUNK), jnp.int32),
        pltpu.VMEM((CPT, CHUNK), jnp.float32),
        pltpu.VMEM((NB, CHUNK, HW), jnp.float32),
        pltpu.VMEM((RPT, HW), jnp.float32),
        pltpu.VMEM_SHARED((NP, HW), jnp.float32),
        pltpu.VMEM_SHARED((NP, HW), jnp.float32),
        [pltpu.SemaphoreType.DMA] * NB,
        [pltpu.SemaphoreType.DMA] * NB,
    ],
)


def kernel(features, edge_index, edge_vals, W0, W1):
    h = _mlp(features, W0, W1)
    ei3 = edge_index.reshape(2, NCHUNKS, CHUNK)
    vals2 = edge_vals.reshape(NCHUNKS, CHUNK)
    zeros = jnp.zeros((RPT, HW), jnp.float32)
    return _spmm(h, ei3, vals2, zeros)


# speedup vs baseline: 1.2722x; 1.2722x over previous
import jax
import jax.numpy as jnp
from jax import lax
from jax.experimental import pallas as pl
from jax.experimental.pallas import tpu as pltpu
from jax.experimental.pallas import tpu_sc as plsc

N = 10000
E = 320000
D_IN = 128
D_OUT = 40
HW = 24
DP = 2 * HW
NC, NS, L = 2, 16, 16
CHUNK = 128
NB = 4
NCHUNKS = E // CHUNK
CPT = 157
CPT_LAST = NCHUNKS - 15 * CPT
NP = 10240
RPT = NP // NS
RLAST = N - 15 * RPT


def _mlp_body(x_ref, w0_ref, w1_ref, o_ref):
    h = jnp.maximum(
        jnp.dot(x_ref[...], w0_ref[...], preferred_element_type=jnp.float32), 0.0)
    h = jnp.dot(h, w1_ref[...], preferred_element_type=jnp.float32)
    o_ref[pl.ds(0, N), :] = jnp.pad(h, ((0, 0), (0, DP - D_OUT)))


_mlp = pl.pallas_call(
    _mlp_body,
    out_shape=jax.ShapeDtypeStruct((NP, DP), jnp.float32),
)


_GDN = lax.GatherDimensionNumbers(
    offset_dims=(), collapsed_slice_dims=(0,), start_index_map=(0,))


def _spmm_body(h_hbm, ei_hbm, vals_hbm, zeros_hbm, out_hbm,
               src_v, dst_v, vals_v, rows_v, stage_v, acc, h_s, gsem, ssem):
    cid = lax.axis_index("c")
    sid = lax.axis_index("s")
    sl = pl.ds(sid * RPT, RPT)
    last = sid == NS - 1
    cpt = jnp.where(last, CPT_LAST, CPT)
    cbase = sid * CPT

    iota = lax.iota(jnp.int32, L)
    duo = iota // 8
    pcol = L + iota - 8 * duo

    pltpu.sync_copy(zeros_hbm, stage_v)
    pltpu.sync_copy(stage_v, acc.at[sl])
    pltpu.sync_copy(h_hbm.at[sl, pl.ds(cid * HW, HW)], stage_v)
    pltpu.sync_copy(stage_v, h_s.at[sl])

    @pl.when(jnp.logical_not(last))
    def _():
        csl = pl.ds(cbase, CPT)
        pltpu.sync_copy(ei_hbm.at[0].at[csl], src_v)
        pltpu.sync_copy(ei_hbm.at[1].at[csl], dst_v)
        pltpu.sync_copy(vals_hbm.at[csl], vals_v)

    @pl.when(last)
    def _():
        csl = pl.ds(cbase, CPT_LAST)
        tsl = pl.ds(0, CPT_LAST)
        pltpu.sync_copy(ei_hbm.at[0].at[csl], src_v.at[tsl])
        pltpu.sync_copy(ei_hbm.at[1].at[csl], dst_v.at[tsl])
        pltpu.sync_copy(vals_hbm.at[csl], vals_v.at[tsl])

    def spmm_round():
        plsc.subcore_barrier()

        for b in range(NB):
            pltpu.async_copy(h_s.at[src_v.at[b]], rows_v.at[b], gsem[b])

        def outer(g, carry):
            for b in range(NB):
                j = g * NB + b

                @pl.when(j < cpt)
                def _():
                    pltpu.make_async_copy(
                        h_s.at[src_v.at[j]], rows_v.at[b], gsem[b]).wait()

                    @plsc.parallel_loop(0, CHUNK, L, unroll=4)
                    def _scale(c0):
                        v16 = vals_v[j, pl.ds(c0, L)]
                        for r in range(L):
                            bc = lax.gather(
                                v16, jnp.full((L, 1), r, jnp.int32), _GDN,
                                slice_sizes=(1,),
                                mode=lax.GatherScatterMode.PROMISE_IN_BOUNDS)
                            seg = rows_v[b, c0 + r, pl.ds(0, L)]
                            rows_v[b, c0 + r, pl.ds(0, L)] = seg * bc
                        for t in range(8):
                            r_idx = (jnp.full((L,), c0 + 2 * t, jnp.int32)
                                     + duo)
                            b_idx = jnp.full((L,), b, jnp.int32)
                            vv = plsc.load_gather(
                                vals_v, [jnp.full((L,), j, jnp.int32), r_idx])
                            seg = plsc.load_gather(
                                rows_v, [b_idx, r_idx, pcol])
                            plsc.store_scatter(rows_v, [b_idx, r_idx, pcol],
                                               seg * vv)

                    pltpu.async_copy(rows_v.at[b], acc.at[dst_v.at[j]],
                                     ssem[b], add=True)

                    b2 = (b - 1) % NB
                    j2 = j - 1
                    jn = j2 + NB

                    @pl.when(j2 >= 0)
                    def _():
                        pltpu.make_async_copy(
                            rows_v.at[b2], acc.at[dst_v.at[j2]],
                            ssem[b2]).wait()

                        @pl.when(jn < cpt)
                        def _():
                            pltpu.async_copy(
                                h_s.at[src_v.at[jn]], rows_v.at[b2],
                                gsem[b2])
            return carry

        lax.fori_loop(0, (CPT + NB - 1) // NB, outer, 0)

        pltpu.make_async_copy(
            rows_v.at[0], acc.at[dst_v.at[cpt - 1]], ssem[0]).wait()

        plsc.subcore_barrier()

    spmm_round()

    pltpu.sync_copy(acc.at[sl], stage_v)
    pltpu.sync_copy(stage_v, h_s.at[sl])
    pltpu.sync_copy(zeros_hbm, stage_v)
    pltpu.sync_copy(stage_v, acc.at[sl])

    spmm_round()

    pltpu.sync_copy(acc.at[sl], stage_v)
    nd = D_OUT - HW

    @pl.when(jnp.logical_not(last))
    def _():
        rs = pl.ds(sid * RPT, RPT)

        @pl.when(cid == 0)
        def _():
            pltpu.sync_copy(stage_v, out_hbm.at[rs, pl.ds(0, HW)])

        @pl.when(cid == 1)
        def _():
            pltpu.sync_copy(stage_v.at[:, pl.ds(0, nd)],
                            out_hbm.at[rs, pl.ds(HW, nd)])

    @pl.when(last)
    def _():
        rs = pl.ds(sid * RPT, RLAST)

        @pl.when(cid == 0)
        def _():
            pltpu.sync_copy(stage_v.at[pl.ds(0, RLAST)],
                            out_hbm.at[rs, pl.ds(0, HW)])

        @pl.when(cid == 1)
        def _():
            pltpu.sync_copy(stage_v.at[pl.ds(0, RLAST), pl.ds(0, nd)],
                            out_hbm.at[rs, pl.ds(HW, nd)])


_spmm = pl.kernel(
    _spmm_body,
    out_type=jax.ShapeDtypeStruct((N, D_OUT), jnp.float32),
    mesh=plsc.VectorSubcoreMesh(core_axis_name="c", subcore_axis_name="s"),
    compiler_params=pltpu.CompilerParams(needs_layout_passes=False,
                                         use_tc_tiling_on_sc=False),
    scratch_types=[
        pltpu.VMEM((CPT, CHUNK), jnp.int32),
        pltpu.VMEM((CPT, CHUNK), jnp.int32),
        pltpu.VMEM((CPT, CHUNK), jnp.float32),
        pltpu.VMEM((NB, CHUNK, HW), jnp.float32),
        pltpu.VMEM((RPT, HW), jnp.float32),
        pltpu.VMEM_SHARED((NP, HW), jnp.float32),
        pltpu.VMEM_SHARED((NP, HW), jnp.float32),
        [pltpu.SemaphoreType.DMA] * NB,
        [pltpu.SemaphoreType.DMA] * NB,
    ],
)


def kernel(features, edge_index, edge_vals, W0, W1):
    h = _mlp(features, W0, W1)
    ei3 = edge_index.reshape(2, NCHUNKS, CHUNK)
    vals2 = edge_vals.reshape(NCHUNKS, CHUNK)
    zeros = jnp.zeros((RPT, HW), jnp.float32)
    return _spmm(h, ei3, vals2, zeros)


# speedup vs baseline: 1.3267x; 1.0428x over previous
import jax
import jax.numpy as jnp
from jax import lax
from jax.experimental import pallas as pl
from jax.experimental.pallas import tpu as pltpu
from jax.experimental.pallas import tpu_sc as plsc

N = 10000
E = 320000
D_IN = 128
D_OUT = 40
HW = 24
DP = 2 * HW
NC, NS, L = 2, 16, 16
CHUNK = 128
NB = 4
NCHUNKS = E // CHUNK
CPT = 157
CPT_LAST = NCHUNKS - 15 * CPT
NP = 10240
RPT = NP // NS
RLAST = N - 15 * RPT


def _mlp_body(x_ref, w0_ref, w1_ref, o_ref):
    h = jnp.maximum(
        jnp.dot(x_ref[...], w0_ref[...], preferred_element_type=jnp.float32), 0.0)
    h = jnp.dot(h, w1_ref[...], preferred_element_type=jnp.float32)
    o_ref[pl.ds(0, N), :] = jnp.pad(h, ((0, 0), (0, DP - D_OUT)))


_mlp = pl.pallas_call(
    _mlp_body,
    out_shape=jax.ShapeDtypeStruct((NP, DP), jnp.float32),
)


_GDN = lax.GatherDimensionNumbers(
    offset_dims=(), collapsed_slice_dims=(0,), start_index_map=(0,))


def _spmm_body(h_hbm, ei_hbm, vals_hbm, zeros_hbm, out_hbm,
               src_v, dst_v, vals_v, rows_v, stage_v, acc, h_s, gsem, ssem):
    cid = lax.axis_index("c")
    sid = lax.axis_index("s")
    sl = pl.ds(sid * RPT, RPT)
    last = sid == NS - 1
    cpt = jnp.where(last, CPT_LAST, CPT)
    cbase = sid * CPT

    iota = lax.iota(jnp.int32, L)
    duo = iota // 8
    pcol = L + iota - 8 * duo

    pltpu.sync_copy(zeros_hbm, stage_v)
    pltpu.sync_copy(stage_v, acc.at[sl])
    pltpu.sync_copy(h_hbm.at[sl, pl.ds(cid * HW, HW)], stage_v)
    pltpu.sync_copy(stage_v, h_s.at[sl])

    @pl.when(jnp.logical_not(last))
    def _():
        csl = pl.ds(cbase, CPT)
        pltpu.sync_copy(ei_hbm.at[0].at[csl], src_v)
        pltpu.sync_copy(ei_hbm.at[1].at[csl], dst_v)
        pltpu.sync_copy(vals_hbm.at[csl], vals_v)

    @pl.when(last)
    def _():
        csl = pl.ds(cbase, CPT_LAST)
        tsl = pl.ds(0, CPT_LAST)
        pltpu.sync_copy(ei_hbm.at[0].at[csl], src_v.at[tsl])
        pltpu.sync_copy(ei_hbm.at[1].at[csl], dst_v.at[tsl])
        pltpu.sync_copy(vals_hbm.at[csl], vals_v.at[tsl])

    def spmm_round():
        plsc.subcore_barrier()

        for b in range(NB):
            pltpu.async_copy(h_s.at[src_v.at[b]], rows_v.at[b], gsem[b])

        def outer(g, carry):
            for b in range(NB):
                j = g * NB + b

                @pl.when(j < cpt)
                def _():
                    pltpu.make_async_copy(
                        h_s.at[src_v.at[j]], rows_v.at[b], gsem[b]).wait()

                    @plsc.parallel_loop(0, CHUNK, L, unroll=1)
                    def _scale(c0):
                        v16 = vals_v[j, pl.ds(c0, L)]
                        for r in range(L):
                            bc = lax.gather(
                                v16, jnp.full((L, 1), r, jnp.int32), _GDN,
                                slice_sizes=(1,),
                                mode=lax.GatherScatterMode.PROMISE_IN_BOUNDS)
                            seg = rows_v[b, c0 + r, pl.ds(0, L)]
                            rows_v[b, c0 + r, pl.ds(0, L)] = seg * bc
                        for t in range(8):
                            r_idx = (jnp.full((L,), c0 + 2 * t, jnp.int32)
                                     + duo)
                            b_idx = jnp.full((L,), b, jnp.int32)
                            vv = plsc.load_gather(
                                vals_v, [jnp.full((L,), j, jnp.int32), r_idx])
                            seg = plsc.load_gather(
                                rows_v, [b_idx, r_idx, pcol])
                            plsc.store_scatter(rows_v, [b_idx, r_idx, pcol],
                                               seg * vv)

                    pltpu.async_copy(rows_v.at[b], acc.at[dst_v.at[j]],
                                     ssem[b], add=True)

                    b2 = (b - 1) % NB
                    j2 = j - 1
                    jn = j2 + NB

                    @pl.when(j2 >= 0)
                    def _():
                        pltpu.make_async_copy(
                            rows_v.at[b2], acc.at[dst_v.at[j2]],
                            ssem[b2]).wait()

                        @pl.when(jn < cpt)
                        def _():
                            pltpu.async_copy(
                                h_s.at[src_v.at[jn]], rows_v.at[b2],
                                gsem[b2])
            return carry

        lax.fori_loop(0, (CPT + NB - 1) // NB, outer, 0)

        pltpu.make_async_copy(
            rows_v.at[0], acc.at[dst_v.at[cpt - 1]], ssem[0]).wait()

        plsc.subcore_barrier()

    spmm_round()

    pltpu.sync_copy(acc.at[sl], stage_v)
    pltpu.sync_copy(stage_v, h_s.at[sl])
    pltpu.sync_copy(zeros_hbm, stage_v)
    pltpu.sync_copy(stage_v, acc.at[sl])

    spmm_round()

    pltpu.sync_copy(acc.at[sl], stage_v)
    nd = D_OUT - HW

    @pl.when(jnp.logical_not(last))
    def _():
        rs = pl.ds(sid * RPT, RPT)

        @pl.when(cid == 0)
        def _():
            pltpu.sync_copy(stage_v, out_hbm.at[rs, pl.ds(0, HW)])

        @pl.when(cid == 1)
        def _():
            pltpu.sync_copy(stage_v.at[:, pl.ds(0, nd)],
                            out_hbm.at[rs, pl.ds(HW, nd)])

    @pl.when(last)
    def _():
        rs = pl.ds(sid * RPT, RLAST)

        @pl.when(cid == 0)
        def _():
            pltpu.sync_copy(stage_v.at[pl.ds(0, RLAST)],
                            out_hbm.at[rs, pl.ds(0, HW)])

        @pl.when(cid == 1)
        def _():
            pltpu.sync_copy(stage_v.at[pl.ds(0, RLAST), pl.ds(0, nd)],
                            out_hbm.at[rs, pl.ds(HW, nd)])


_spmm = pl.kernel(
    _spmm_body,
    out_type=jax.ShapeDtypeStruct((N, D_OUT), jnp.float32),
    mesh=plsc.VectorSubcoreMesh(core_axis_name="c", subcore_axis_name="s"),
    compiler_params=pltpu.CompilerParams(needs_layout_passes=False,
                                         use_tc_tiling_on_sc=False),
    scratch_types=[
        pltpu.VMEM((CPT, CHUNK), jnp.int32),
        pltpu.VMEM((CPT, CHUNK), jnp.int32),
        pltpu.VMEM((CPT, CHUNK), jnp.float32),
        pltpu.VMEM((NB, CHUNK, HW), jnp.float32),
        pltpu.VMEM((RPT, HW), jnp.float32),
        pltpu.VMEM_SHARED((NP, HW), jnp.float32),
        pltpu.VMEM_SHARED((NP, HW), jnp.float32),
        [pltpu.SemaphoreType.DMA] * NB,
        [pltpu.SemaphoreType.DMA] * NB,
    ],
)


def kernel(features, edge_index, edge_vals, W0, W1):
    h = _mlp(features, W0, W1)
    ei3 = edge_index.reshape(2, NCHUNKS, CHUNK)
    vals2 = edge_vals.reshape(NCHUNKS, CHUNK)
    zeros = jnp.zeros((RPT, HW), jnp.float32)
    return _spmm(h, ei3, vals2, zeros)
